# baseline (speedup 1.0000x reference)
import functools

import jax
import jax.numpy as jnp
from jax.experimental import pallas as pl
from jax.experimental.pallas import tpu as pltpu

N = 2048
E = 1024
H = 16
DH = 64
TOPK = 64
_INV_SCALE = 1.0 / (DH ** 0.5)
_RB = 256
_PB = 512


def _proj_body(x_ref, wq_ref, bq_ref, wk_ref, bk_ref, q_ref, k_ref):
    x = x_ref[...]
    dn = (((1,), (1,)), ((), ()))
    q_ref[...] = jax.lax.dot_general(
        x, wq_ref[...], dn, preferred_element_type=jnp.float32) + bq_ref[...]
    k_ref[...] = jax.lax.dot_general(
        x, wk_ref[...], dn, preferred_element_type=jnp.float32) + bk_ref[...]


def _fold(x):
    i = jax.lax.bitcast_convert_type(x, jnp.int32)
    return jnp.where(i < 0, i ^ jnp.int32(0x7FFFFFFF), i)


def _unfold(kk):
    return jax.lax.bitcast_convert_type(
        jnp.where(kk < 0, kk ^ jnp.int32(0x7FFFFFFF), kk), jnp.float32)


_KEY_PINF = 2139095040
_KEY_NINF = -2139095041
_Z_INIT = 1.8627
_SLOPE = 1.0 / (N * 0.07036)


def _attn_body(q_ref, k_ref, o_ref):
    q = q_ref[0]
    k = k_ref[0]
    dn = (((1,), (1,)), ((), ()))
    sim = jax.lax.dot_general(
        q, k, dn, preferred_element_type=jnp.float32) * _INV_SCALE
    rb = q.shape[0]

    def count(c_f):
        return jnp.sum((sim >= c_f).astype(jnp.int32), axis=1, keepdims=True)

    def update(state, candk):
        lok, cntlo, hik, cnthi = state
        cnt = count(_unfold(candk))
        take = cnt >= TOPK
        return (jnp.where(take, candk, lok), jnp.where(take, cnt, cntlo),
                jnp.where(take, hik, candk), jnp.where(take, cnthi, cnt))

    kbar = jnp.sum(k, axis=0, keepdims=True) * (1.0 / N)
    mu = jnp.sum(q * kbar, axis=1, keepdims=True) * _INV_SCALE
    g = jax.lax.dot_general(k, k, (((0,), (0,)), ((), ())),
                            preferred_element_type=jnp.float32)
    a = jax.lax.dot_general(q, g, (((1,), (0,)), ((), ())),
                            preferred_element_type=jnp.float32)
    e2 = jnp.sum(a * q, axis=1, keepdims=True) * (_INV_SCALE * _INV_SCALE / N)
    sig = jnp.sqrt(jnp.maximum(e2 - mu * mu, 0.0))

    lok = jnp.full((rb, 1), jnp.int32(_KEY_NINF))
    cntlo = jnp.full((rb, 1), jnp.int32(N))
    hik = jnp.full((rb, 1), jnp.int32(_KEY_PINF))
    cnthi = jnp.zeros((rb, 1), jnp.int32)
    state = (lok, cntlo, hik, cnthi)

    c0 = mu + _Z_INIT * sig
    k0 = jnp.clip(_fold(c0), lok + 1, hik - 1)
    cnt0 = count(_unfold(k0))
    take0 = cnt0 >= TOPK
    state = (jnp.where(take0, k0, lok), jnp.where(take0, cnt0, cntlo),
             jnp.where(take0, hik, k0), jnp.where(take0, cnthi, cnt0))
    c1 = _unfold(k0) + (cnt0 - TOPK).astype(jnp.float32) * sig * _SLOPE
    done = state[1] == TOPK
    k1 = jnp.clip(_fold(c1), state[0] + 1, state[2] - 1)
    state = update(state, jnp.where(done, state[0], k1))

    def cond(carry):
        (lok, cntlo, hik, _), it = carry
        live = jnp.logical_and(cntlo != TOPK, (lok + 1) < hik)
        return jnp.logical_and(it < 64, jnp.any(live))

    def body(carry):
        (lok, cntlo, hik, cnthi), it = carry
        lo_f, hi_f = _unfold(lok), _unfold(hik)
        frac = (cntlo - TOPK).astype(jnp.float32) / (
            (cntlo - cnthi).astype(jnp.float32))
        interpk = _fold(lo_f + (hi_f - lo_f) * frac)
        midk = lok + ((hik >> 1) - (lok >> 1))
        candk = jnp.where((it & 1) == 1, midk, interpk)
        candk = jnp.clip(candk, lok + 1, hik - 1)
        dn_row = jnp.logical_or(cntlo == TOPK, (lok + 1) >= hik)
        candk = jnp.where(dn_row, lok, candk)
        return update((lok, cntlo, hik, cnthi), candk), it + 1

    (lok, _, _, _), _ = jax.lax.while_loop(cond, body, (state, jnp.int32(0)))

    o_ref[0] = jnp.where(sim >= _unfold(lok), sim, 0.0)


@jax.jit
def kernel(embedding_matrix, Wq, bq, Wk, bk):
    x = embedding_matrix
    q, k = pl.pallas_call(
        _proj_body,
        grid=(N // _PB,),
        in_specs=[
            pl.BlockSpec((_PB, E), lambda i: (i, 0)),
            pl.BlockSpec((E, E), lambda i: (0, 0)),
            pl.BlockSpec((1, E), lambda i: (0, 0)),
            pl.BlockSpec((E, E), lambda i: (0, 0)),
            pl.BlockSpec((1, E), lambda i: (0, 0)),
        ],
        out_specs=[
            pl.BlockSpec((_PB, E), lambda i: (i, 0)),
            pl.BlockSpec((_PB, E), lambda i: (i, 0)),
        ],
        out_shape=[
            jax.ShapeDtypeStruct((N, E), jnp.float32),
            jax.ShapeDtypeStruct((N, E), jnp.float32),
        ],
    )(x, Wq, bq.reshape(1, E), Wk, bk.reshape(1, E))

    qh = q.reshape(N, H, DH).transpose(1, 0, 2)
    kh = k.reshape(N, H, DH).transpose(1, 0, 2)

    out = pl.pallas_call(
        _attn_body,
        grid=(H, N // _RB),
        in_specs=[
            pl.BlockSpec((1, _RB, DH), lambda h, i: (h, i, 0)),
            pl.BlockSpec((1, N, DH), lambda h, i: (h, 0, 0)),
        ],
        out_specs=pl.BlockSpec((1, _RB, N), lambda h, i: (h, i, 0)),
        out_shape=jax.ShapeDtypeStruct((H, N, N), jnp.float32),
    )(qh, kh)
    return out


# speedup vs baseline: 2.1806x; 2.1806x over previous
import functools

import jax
import jax.numpy as jnp
from jax.experimental import pallas as pl
from jax.experimental.pallas import tpu as pltpu

N = 2048
E = 1024
H = 16
DH = 64
TOPK = 64
_INV_SCALE = 1.0 / (DH ** 0.5)
_RB = 256
_PB = 512


def _proj_body(x_ref, wq_ref, bq_ref, wk_ref, bk_ref, q_ref, k_ref):
    x = x_ref[...]
    dn = (((1,), (1,)), ((), ()))
    q_ref[...] = jax.lax.dot_general(
        x, wq_ref[...], dn, preferred_element_type=jnp.float32) + bq_ref[...]
    k_ref[...] = jax.lax.dot_general(
        x, wk_ref[...], dn, preferred_element_type=jnp.float32) + bk_ref[...]


def _fold(x):
    i = jax.lax.bitcast_convert_type(x, jnp.int32)
    return jnp.where(i < 0, i ^ jnp.int32(0x7FFFFFFF), i)


def _unfold(kk):
    return jax.lax.bitcast_convert_type(
        jnp.where(kk < 0, kk ^ jnp.int32(0x7FFFFFFF), kk), jnp.float32)


_KEY_PINF = 2139095040
_KEY_NINF = -2139095041
_Z_INIT = 1.8627
_SLOPE = 1.0 / (N * 0.07036)
_DESC = 2


def _attn_body(q_ref, k_ref, o_ref):
    q = q_ref[0]
    k = k_ref[0]
    dn = (((1,), (1,)), ((), ()))
    sim = jax.lax.dot_general(
        q, k, dn, preferred_element_type=jnp.float32) * _INV_SCALE
    rb = q.shape[0]

    def count(c_f):
        return jnp.sum((sim >= c_f).astype(jnp.int32), axis=1, keepdims=True)

    def update(state, candk):
        lok, cntlo, hik, cnthi = state
        cnt = count(_unfold(candk))
        take = cnt >= TOPK
        return (jnp.where(take, candk, lok), jnp.where(take, cnt, cntlo),
                jnp.where(take, hik, candk), jnp.where(take, cnthi, cnt))

    kbar = jnp.sum(k, axis=0, keepdims=True) * (1.0 / N)
    mu = jnp.sum(q * kbar, axis=1, keepdims=True) * _INV_SCALE
    g = jax.lax.dot_general(k, k, (((0,), (0,)), ((), ())),
                            preferred_element_type=jnp.float32)
    a = jax.lax.dot_general(q, g, (((1,), (0,)), ((), ())),
                            preferred_element_type=jnp.float32)
    e2 = jnp.sum(a * q, axis=1, keepdims=True) * (_INV_SCALE * _INV_SCALE / N)
    sig = jnp.sqrt(jnp.maximum(e2 - mu * mu, 0.0))

    lok = jnp.full((rb, 1), jnp.int32(_KEY_NINF))
    cntlo = jnp.full((rb, 1), jnp.int32(N))
    hik = jnp.full((rb, 1), jnp.int32(_KEY_PINF))
    cnthi = jnp.zeros((rb, 1), jnp.int32)
    state = (lok, cntlo, hik, cnthi)

    c0 = mu + _Z_INIT * sig
    k0 = jnp.clip(_fold(c0), lok + 1, hik - 1)
    cnt0 = count(_unfold(k0))
    take0 = cnt0 >= TOPK
    state = (jnp.where(take0, k0, lok), jnp.where(take0, cnt0, cntlo),
             jnp.where(take0, hik, k0), jnp.where(take0, cnthi, cnt0))
    tgt = jnp.where(cnt0 < TOPK, jnp.int32(TOPK + 40), jnp.int32(TOPK - 28))
    c1 = _unfold(k0) + (cnt0 - tgt).astype(jnp.float32) * sig * _SLOPE
    done = state[1] == TOPK
    k1 = jnp.clip(_fold(c1), state[0] + 1, state[2] - 1)
    state = update(state, jnp.where(done, state[0], k1))

    def live_mask(st):
        lok, cntlo, hik, cnthi = st
        return ((cntlo != TOPK) & (TOPK - cnthi > _DESC) & ((lok + 1) < hik))

    def falsi_body(i, st):
        lok, cntlo, hik, cnthi = st
        lo_f, hi_f = _unfold(lok), _unfold(hik)
        frac = (cntlo - (TOPK - 1)).astype(jnp.float32) / (
            (cntlo - cnthi).astype(jnp.float32))
        interpk = _fold(lo_f + (hi_f - lo_f) * jnp.clip(frac, 0.0, 1.0))
        midk = lok + ((hik >> 1) - (lok >> 1))
        candk = jnp.where((i % 3) == 2, midk, interpk)
        candk = jnp.clip(candk, lok + 1, hik - 1)
        candk = jnp.where(live_mask(st), candk, lok)
        return update(st, candk)

    state = jax.lax.fori_loop(0, 6, falsi_body, state)

    def safe_cond(carry):
        st, it = carry
        return jnp.logical_and(it < 64, jnp.any(live_mask(st)))

    def safe_body(carry):
        st, it = carry
        lok, cntlo, hik, cnthi = st
        midk = jnp.clip(lok + ((hik >> 1) - (lok >> 1)), lok + 1, hik - 1)
        candk = jnp.where(live_mask(st), midk, lok)
        return update(st, candk), it + 1

    state, _ = jax.lax.while_loop(safe_cond, safe_body, (state, jnp.int32(0)))

    lok, cntlo, hik, cnthi = state
    tf = _unfold(lok)
    dni = (cntlo == TOPK).astype(jnp.int32)
    hif = _unfold(hik)

    def desc_cond(carry):
        _, dni, _, it = carry
        return jnp.logical_and(it < 64, jnp.any(dni == 0))

    def desc_body(carry):
        tf, dni, hif, it = carry
        dn = dni > 0
        m = jnp.max(jnp.where(sim < hif, sim, jnp.float32(-3.4e38)),
                    axis=1, keepdims=True)
        cntm = count(m)
        fin = (~dn) & (cntm >= TOPK)
        tf = jnp.where(fin, m, tf)
        dn = dn | fin
        adv = (~dn) & (cntm < TOPK)
        hif = jnp.where(adv, m, hif)
        return tf, dn.astype(jnp.int32), hif, it + 1

    tf, _, _, _ = jax.lax.while_loop(
        desc_cond, desc_body, (tf, dni, hif, jnp.int32(0)))

    o_ref[0] = jnp.where(sim >= tf, sim, 0.0)


@jax.jit
def kernel(embedding_matrix, Wq, bq, Wk, bk):
    x = embedding_matrix
    q, k = pl.pallas_call(
        _proj_body,
        grid=(N // _PB,),
        in_specs=[
            pl.BlockSpec((_PB, E), lambda i: (i, 0)),
            pl.BlockSpec((E, E), lambda i: (0, 0)),
            pl.BlockSpec((1, E), lambda i: (0, 0)),
            pl.BlockSpec((E, E), lambda i: (0, 0)),
            pl.BlockSpec((1, E), lambda i: (0, 0)),
        ],
        out_specs=[
            pl.BlockSpec((_PB, E), lambda i: (i, 0)),
            pl.BlockSpec((_PB, E), lambda i: (i, 0)),
        ],
        out_shape=[
            jax.ShapeDtypeStruct((N, E), jnp.float32),
            jax.ShapeDtypeStruct((N, E), jnp.float32),
        ],
    )(x, Wq, bq.reshape(1, E), Wk, bk.reshape(1, E))

    qh = q.reshape(N, H, DH).transpose(1, 0, 2)
    kh = k.reshape(N, H, DH).transpose(1, 0, 2)

    out = pl.pallas_call(
        _attn_body,
        grid=(H, N // _RB),
        in_specs=[
            pl.BlockSpec((1, _RB, DH), lambda h, i: (h, i, 0)),
            pl.BlockSpec((1, N, DH), lambda h, i: (h, 0, 0)),
        ],
        out_specs=pl.BlockSpec((1, _RB, N), lambda h, i: (h, i, 0)),
        out_shape=jax.ShapeDtypeStruct((H, N, N), jnp.float32),
    )(qh, kh)
    return out


# speedup vs baseline: 2.4729x; 1.1341x over previous
import functools

import jax
import jax.numpy as jnp
from jax.experimental import pallas as pl
from jax.experimental.pallas import tpu as pltpu

N = 2048
E = 1024
H = 16
DH = 64
TOPK = 64
_INV_SCALE = 1.0 / (DH ** 0.5)
_RB = 256
_PB = 512


def _proj_body(x_ref, wq_ref, bq_ref, wk_ref, bk_ref, q_ref, k_ref):
    x = x_ref[...]
    dn = (((1,), (1,)), ((), ()))
    q_ref[...] = jax.lax.dot_general(
        x, wq_ref[...], dn, preferred_element_type=jnp.float32) + bq_ref[...]
    k_ref[...] = jax.lax.dot_general(
        x, wk_ref[...], dn, preferred_element_type=jnp.float32) + bk_ref[...]


def _fold(x):
    i = jax.lax.bitcast_convert_type(x, jnp.int32)
    return jnp.where(i < 0, i ^ jnp.int32(0x7FFFFFFF), i)


def _unfold(kk):
    return jax.lax.bitcast_convert_type(
        jnp.where(kk < 0, kk ^ jnp.int32(0x7FFFFFFF), kk), jnp.float32)


_KEY_PINF = 2139095040
_KEY_NINF = -2139095041
_Z_INIT = 1.8627
_SLOPE = 1.0 / (N * 0.07036)
_ASC = 3
_FRAC_TGT = float(TOPK + 2)


def _attn_body(q_ref, k_ref, o_ref):
    q = q_ref[0]
    k = k_ref[0]
    dn = (((1,), (1,)), ((), ()))
    sim = jax.lax.dot_general(
        q, k, dn, preferred_element_type=jnp.float32) * _INV_SCALE
    rb = q.shape[0]

    def count(c_f):
        return jnp.sum((sim >= c_f).astype(jnp.int32), axis=1, keepdims=True)

    def update(state, candk):
        lok, cntlo, hik, cnthi = state
        cnt = count(_unfold(candk))
        take = cnt >= TOPK
        return (jnp.where(take, candk, lok), jnp.where(take, cnt, cntlo),
                jnp.where(take, hik, candk), jnp.where(take, cnthi, cnt))

    kbar = jnp.sum(k, axis=0, keepdims=True) * (1.0 / N)
    mu = jnp.sum(q * kbar, axis=1, keepdims=True) * _INV_SCALE
    g = jax.lax.dot_general(k, k, (((0,), (0,)), ((), ())),
                            preferred_element_type=jnp.float32)
    a = jax.lax.dot_general(q, g, (((1,), (0,)), ((), ())),
                            preferred_element_type=jnp.float32)
    e2 = jnp.sum(a * q, axis=1, keepdims=True) * (_INV_SCALE * _INV_SCALE / N)
    sig = jnp.sqrt(jnp.maximum(e2 - mu * mu, 0.0))

    lok = jnp.full((rb, 1), jnp.int32(_KEY_NINF))
    cntlo = jnp.full((rb, 1), jnp.int32(N))
    hik = jnp.full((rb, 1), jnp.int32(_KEY_PINF))
    cnthi = jnp.zeros((rb, 1), jnp.int32)
    state = (lok, cntlo, hik, cnthi)

    c0 = mu + _Z_INIT * sig
    k0 = jnp.clip(_fold(c0), lok + 1, hik - 1)
    cnt0 = count(_unfold(k0))
    take0 = cnt0 >= TOPK
    state = (jnp.where(take0, k0, lok), jnp.where(take0, cnt0, cntlo),
             jnp.where(take0, hik, k0), jnp.where(take0, cnthi, cnt0))
    tgt = jnp.where(cnt0 < TOPK, jnp.int32(TOPK + 12), jnp.int32(TOPK - 8))
    c1 = _unfold(k0) + (cnt0 - tgt).astype(jnp.float32) * sig * _SLOPE
    done = state[1] == TOPK
    k1 = jnp.clip(_fold(c1), state[0] + 1, state[2] - 1)
    state = update(state, jnp.where(done, state[0], k1))

    def live_mask(st):
        lok, cntlo, hik, cnthi = st
        return ((cntlo != TOPK) & (cntlo - TOPK > _ASC) & ((lok + 1) < hik))

    def falsi_body(i, st):
        lok, cntlo, hik, cnthi = st
        lo_f, hi_f = _unfold(lok), _unfold(hik)
        frac = (cntlo.astype(jnp.float32) - _FRAC_TGT) / (
            (cntlo - cnthi).astype(jnp.float32))
        interpk = _fold(lo_f + (hi_f - lo_f) * jnp.clip(frac, 0.0, 1.0))
        midk = lok + ((hik >> 1) - (lok >> 1))
        candk = jnp.where((i % 3) == 2, midk, interpk)
        candk = jnp.clip(candk, lok + 1, hik - 1)
        candk = jnp.where(live_mask(st), candk, lok)
        return update(st, candk)

    state = jax.lax.fori_loop(0, 5, falsi_body, state)

    def safe_cond(carry):
        st, it = carry
        return jnp.logical_and(it < 64, jnp.any(live_mask(st)))

    def safe_body(carry):
        st, it = carry
        lok, cntlo, hik, cnthi = st
        midk = jnp.clip(lok + ((hik >> 1) - (lok >> 1)), lok + 1, hik - 1)
        candk = jnp.where(live_mask(st), midk, lok)
        return update(st, candk), it + 1

    state, _ = jax.lax.while_loop(safe_cond, safe_body, (state, jnp.int32(0)))

    lok, cntlo, hik, cnthi = state
    rem = jnp.where(cntlo == TOPK, jnp.int32(0), cntlo - TOPK)
    m = _unfold(lok - 1)

    def asc_cond(carry):
        _, rem, it = carry
        return jnp.logical_and(it < 256, jnp.any(rem > 0))

    def asc_body(carry):
        m, rem, it = carry
        act = rem > 0
        mnew = jnp.min(jnp.where(sim > m, sim, jnp.float32(3.4e38)),
                       axis=1, keepdims=True)
        m = jnp.where(act, mnew, m)
        rem = jnp.where(act, rem - 1, rem)
        return m, rem, it + 1

    m, _, _ = jax.lax.while_loop(
        asc_cond, asc_body, (m, rem, jnp.int32(0)))

    tf = jnp.where(cntlo == TOPK, _unfold(lok), _unfold(_fold(m) + 1))
    cnt_v = count(tf)
    dni = (cnt_v >= TOPK).astype(jnp.int32)
    hif = tf

    def rep_cond(carry):
        _, dni, _, it = carry
        return jnp.logical_and(it < 64, jnp.any(dni == 0))

    def rep_body(carry):
        tf, dni, hif, it = carry
        dn = dni > 0
        m2 = jnp.max(jnp.where(sim < hif, sim, jnp.float32(-3.4e38)),
                     axis=1, keepdims=True)
        cnt2 = count(m2)
        fin = (~dn) & (cnt2 >= TOPK)
        tf = jnp.where(fin, m2, tf)
        dn = dn | fin
        adv = (~dn) & (cnt2 < TOPK)
        hif = jnp.where(adv, m2, hif)
        return tf, dn.astype(jnp.int32), hif, it + 1

    tf, _, _, _ = jax.lax.while_loop(
        rep_cond, rep_body, (tf, dni, hif, jnp.int32(0)))

    o_ref[0] = jnp.where(sim >= tf, sim, 0.0)


@jax.jit
def kernel(embedding_matrix, Wq, bq, Wk, bk):
    x = embedding_matrix
    q, k = pl.pallas_call(
        _proj_body,
        grid=(N // _PB,),
        in_specs=[
            pl.BlockSpec((_PB, E), lambda i: (i, 0)),
            pl.BlockSpec((E, E), lambda i: (0, 0)),
            pl.BlockSpec((1, E), lambda i: (0, 0)),
            pl.BlockSpec((E, E), lambda i: (0, 0)),
            pl.BlockSpec((1, E), lambda i: (0, 0)),
        ],
        out_specs=[
            pl.BlockSpec((_PB, E), lambda i: (i, 0)),
            pl.BlockSpec((_PB, E), lambda i: (i, 0)),
        ],
        out_shape=[
            jax.ShapeDtypeStruct((N, E), jnp.float32),
            jax.ShapeDtypeStruct((N, E), jnp.float32),
        ],
    )(x, Wq, bq.reshape(1, E), Wk, bk.reshape(1, E))

    qh = q.reshape(N, H, DH).transpose(1, 0, 2)
    kh = k.reshape(N, H, DH).transpose(1, 0, 2)

    out = pl.pallas_call(
        _attn_body,
        grid=(H, N // _RB),
        in_specs=[
            pl.BlockSpec((1, _RB, DH), lambda h, i: (h, i, 0)),
            pl.BlockSpec((1, N, DH), lambda h, i: (h, 0, 0)),
        ],
        out_specs=pl.BlockSpec((1, _RB, N), lambda h, i: (h, i, 0)),
        out_shape=jax.ShapeDtypeStruct((H, N, N), jnp.float32),
    )(qh, kh)
    return out


# speedup vs baseline: 2.7439x; 1.1096x over previous
import functools

import jax
import jax.numpy as jnp
from jax.experimental import pallas as pl
from jax.experimental.pallas import tpu as pltpu

N = 2048
E = 1024
H = 16
DH = 64
TOPK = 64
_INV_SCALE = 1.0 / (DH ** 0.5)
_RB = 512
_PB = 512


def _proj_body(x_ref, wq_ref, bq_ref, wk_ref, bk_ref, q_ref, k_ref):
    x = x_ref[...]
    dn = (((1,), (1,)), ((), ()))
    q_ref[...] = jax.lax.dot_general(
        x, wq_ref[...], dn, preferred_element_type=jnp.float32) + bq_ref[...]
    k_ref[...] = jax.lax.dot_general(
        x, wk_ref[...], dn, preferred_element_type=jnp.float32) + bk_ref[...]


def _fold(x):
    i = jax.lax.bitcast_convert_type(x, jnp.int32)
    return jnp.where(i < 0, i ^ jnp.int32(0x7FFFFFFF), i)


def _unfold(kk):
    return jax.lax.bitcast_convert_type(
        jnp.where(kk < 0, kk ^ jnp.int32(0x7FFFFFFF), kk), jnp.float32)


_KEY_PINF = 2139095040
_KEY_NINF = -2139095041
_Z_INIT = 1.8627
_SLOPE = 1.0 / (N * 0.07036)
_ASC = 3
_FRAC_TGT = float(TOPK + 2)


def _attn_body(q_ref, k_ref, o_ref):
    q = q_ref[0]
    k = k_ref[0]
    dn = (((1,), (1,)), ((), ()))
    sim = jax.lax.dot_general(
        q, k, dn, preferred_element_type=jnp.float32) * _INV_SCALE
    rb = q.shape[0]

    def count(c_f):
        return jnp.sum((sim >= c_f).astype(jnp.int32), axis=1, keepdims=True)

    def update(state, candk):
        lok, cntlo, hik, cnthi = state
        cnt = count(_unfold(candk))
        take = cnt >= TOPK
        return (jnp.where(take, candk, lok), jnp.where(take, cnt, cntlo),
                jnp.where(take, hik, candk), jnp.where(take, cnthi, cnt))

    kbar = jnp.sum(k, axis=0, keepdims=True) * (1.0 / N)
    mu = jnp.sum(q * kbar, axis=1, keepdims=True) * _INV_SCALE
    g = jax.lax.dot_general(k, k, (((0,), (0,)), ((), ())),
                            preferred_element_type=jnp.float32)
    a = jax.lax.dot_general(q, g, (((1,), (0,)), ((), ())),
                            preferred_element_type=jnp.float32)
    e2 = jnp.sum(a * q, axis=1, keepdims=True) * (_INV_SCALE * _INV_SCALE / N)
    sig = jnp.sqrt(jnp.maximum(e2 - mu * mu, 0.0))

    lok = jnp.full((rb, 1), jnp.int32(_KEY_NINF))
    cntlo = jnp.full((rb, 1), jnp.int32(N))
    hik = jnp.full((rb, 1), jnp.int32(_KEY_PINF))
    cnthi = jnp.zeros((rb, 1), jnp.int32)
    state = (lok, cntlo, hik, cnthi)

    c0 = mu + _Z_INIT * sig
    k0 = jnp.clip(_fold(c0), lok + 1, hik - 1)
    cnt0 = count(_unfold(k0))
    take0 = cnt0 >= TOPK
    state = (jnp.where(take0, k0, lok), jnp.where(take0, cnt0, cntlo),
             jnp.where(take0, hik, k0), jnp.where(take0, cnthi, cnt0))
    tgt = jnp.where(cnt0 < TOPK, jnp.int32(TOPK + 12), jnp.int32(TOPK - 8))
    c1 = _unfold(k0) + (cnt0 - tgt).astype(jnp.float32) * sig * _SLOPE
    done = state[1] == TOPK
    k1 = jnp.clip(_fold(c1), state[0] + 1, state[2] - 1)
    state = update(state, jnp.where(done, state[0], k1))

    def live_mask(st):
        lok, cntlo, hik, cnthi = st
        return ((cntlo != TOPK) & (cntlo - TOPK > _ASC) & ((lok + 1) < hik))

    def falsi_body(i, st):
        lok, cntlo, hik, cnthi = st
        lo_f, hi_f = _unfold(lok), _unfold(hik)
        frac = (cntlo.astype(jnp.float32) - _FRAC_TGT) / (
            (cntlo - cnthi).astype(jnp.float32))
        interpk = _fold(lo_f + (hi_f - lo_f) * jnp.clip(frac, 0.0, 1.0))
        midk = lok + ((hik >> 1) - (lok >> 1))
        candk = jnp.where((i % 3) == 2, midk, interpk)
        candk = jnp.clip(candk, lok + 1, hik - 1)
        candk = jnp.where(live_mask(st), candk, lok)
        return update(st, candk)

    state = jax.lax.fori_loop(0, 5, falsi_body, state)

    def safe_cond(carry):
        st, it = carry
        return jnp.logical_and(it < 64, jnp.any(live_mask(st)))

    def safe_body(carry):
        st, it = carry
        lok, cntlo, hik, cnthi = st
        midk = jnp.clip(lok + ((hik >> 1) - (lok >> 1)), lok + 1, hik - 1)
        candk = jnp.where(live_mask(st), midk, lok)
        return update(st, candk), it + 1

    state, _ = jax.lax.while_loop(safe_cond, safe_body, (state, jnp.int32(0)))

    lok, cntlo, hik, cnthi = state
    rem = jnp.where(cntlo == TOPK, jnp.int32(0), cntlo - TOPK)
    m = _unfold(lok - 1)

    def asc_cond(carry):
        _, rem, it = carry
        return jnp.logical_and(it < 256, jnp.any(rem > 0))

    def asc_body(carry):
        m, rem, it = carry
        act = rem > 0
        mnew = jnp.min(jnp.where(sim > m, sim, jnp.float32(3.4e38)),
                       axis=1, keepdims=True)
        m = jnp.where(act, mnew, m)
        rem = jnp.where(act, rem - 1, rem)
        return m, rem, it + 1

    m, _, _ = jax.lax.while_loop(
        asc_cond, asc_body, (m, rem, jnp.int32(0)))

    tf = jnp.where(cntlo == TOPK, _unfold(lok), _unfold(_fold(m) + 1))
    cnt_v = count(tf)
    dni = (cnt_v >= TOPK).astype(jnp.int32)
    hif = tf

    def rep_cond(carry):
        _, dni, _, it = carry
        return jnp.logical_and(it < 64, jnp.any(dni == 0))

    def rep_body(carry):
        tf, dni, hif, it = carry
        dn = dni > 0
        m2 = jnp.max(jnp.where(sim < hif, sim, jnp.float32(-3.4e38)),
                     axis=1, keepdims=True)
        cnt2 = count(m2)
        fin = (~dn) & (cnt2 >= TOPK)
        tf = jnp.where(fin, m2, tf)
        dn = dn | fin
        adv = (~dn) & (cnt2 < TOPK)
        hif = jnp.where(adv, m2, hif)
        return tf, dn.astype(jnp.int32), hif, it + 1

    tf, _, _, _ = jax.lax.while_loop(
        rep_cond, rep_body, (tf, dni, hif, jnp.int32(0)))

    o_ref[0] = jnp.where(sim >= tf, sim, 0.0)


@jax.jit
def kernel(embedding_matrix, Wq, bq, Wk, bk):
    x = embedding_matrix
    q, k = pl.pallas_call(
        _proj_body,
        grid=(N // _PB,),
        in_specs=[
            pl.BlockSpec((_PB, E), lambda i: (i, 0)),
            pl.BlockSpec((E, E), lambda i: (0, 0)),
            pl.BlockSpec((1, E), lambda i: (0, 0)),
            pl.BlockSpec((E, E), lambda i: (0, 0)),
            pl.BlockSpec((1, E), lambda i: (0, 0)),
        ],
        out_specs=[
            pl.BlockSpec((_PB, E), lambda i: (i, 0)),
            pl.BlockSpec((_PB, E), lambda i: (i, 0)),
        ],
        out_shape=[
            jax.ShapeDtypeStruct((N, E), jnp.float32),
            jax.ShapeDtypeStruct((N, E), jnp.float32),
        ],
    )(x, Wq, bq.reshape(1, E), Wk, bk.reshape(1, E))

    qh = q.reshape(N, H, DH).transpose(1, 0, 2)
    kh = k.reshape(N, H, DH).transpose(1, 0, 2)

    out = pl.pallas_call(
        _attn_body,
        grid=(H, N // _RB),
        in_specs=[
            pl.BlockSpec((1, _RB, DH), lambda h, i: (h, i, 0)),
            pl.BlockSpec((1, N, DH), lambda h, i: (h, 0, 0)),
        ],
        out_specs=pl.BlockSpec((1, _RB, N), lambda h, i: (h, i, 0)),
        out_shape=jax.ShapeDtypeStruct((H, N, N), jnp.float32),
    )(qh, kh)
    return out


# speedup vs baseline: 2.8207x; 1.0280x over previous
import functools

import jax
import jax.numpy as jnp
from jax.experimental import pallas as pl
from jax.experimental.pallas import tpu as pltpu

N = 2048
E = 1024
H = 16
DH = 64
TOPK = 64
_INV_SCALE = 1.0 / (DH ** 0.5)
_RB = 1024
_PB = 512


def _proj_body(x_ref, wq_ref, bq_ref, wk_ref, bk_ref, q_ref, k_ref):
    x = x_ref[...]
    dn = (((1,), (1,)), ((), ()))
    q_ref[...] = jax.lax.dot_general(
        x, wq_ref[...], dn, preferred_element_type=jnp.float32) + bq_ref[...]
    k_ref[...] = jax.lax.dot_general(
        x, wk_ref[...], dn, preferred_element_type=jnp.float32) + bk_ref[...]


def _fold(x):
    i = jax.lax.bitcast_convert_type(x, jnp.int32)
    return jnp.where(i < 0, i ^ jnp.int32(0x7FFFFFFF), i)


def _unfold(kk):
    return jax.lax.bitcast_convert_type(
        jnp.where(kk < 0, kk ^ jnp.int32(0x7FFFFFFF), kk), jnp.float32)


_KEY_PINF = 2139095040
_KEY_NINF = -2139095041
_Z_INIT = 1.8627
_SLOPE = 1.0 / (N * 0.07036)
_ASC = 3
_FRAC_TGT = float(TOPK + 2)


def _attn_body(q_ref, k_ref, o_ref):
    q = q_ref[0]
    k = k_ref[0]
    dn = (((1,), (1,)), ((), ()))
    sim = jax.lax.dot_general(
        q, k, dn, preferred_element_type=jnp.float32) * _INV_SCALE
    rb = q.shape[0]

    def count(c_f):
        return jnp.sum((sim >= c_f).astype(jnp.int32), axis=1, keepdims=True)

    def update(state, candk):
        lok, cntlo, hik, cnthi = state
        cnt = count(_unfold(candk))
        take = cnt >= TOPK
        return (jnp.where(take, candk, lok), jnp.where(take, cnt, cntlo),
                jnp.where(take, hik, candk), jnp.where(take, cnthi, cnt))

    kbar = jnp.sum(k, axis=0, keepdims=True) * (1.0 / N)
    mu = jnp.sum(q * kbar, axis=1, keepdims=True) * _INV_SCALE
    g = jax.lax.dot_general(k, k, (((0,), (0,)), ((), ())),
                            preferred_element_type=jnp.float32)
    a = jax.lax.dot_general(q, g, (((1,), (0,)), ((), ())),
                            preferred_element_type=jnp.float32)
    e2 = jnp.sum(a * q, axis=1, keepdims=True) * (_INV_SCALE * _INV_SCALE / N)
    sig = jnp.sqrt(jnp.maximum(e2 - mu * mu, 0.0))

    lok = jnp.full((rb, 1), jnp.int32(_KEY_NINF))
    cntlo = jnp.full((rb, 1), jnp.int32(N))
    hik = jnp.full((rb, 1), jnp.int32(_KEY_PINF))
    cnthi = jnp.zeros((rb, 1), jnp.int32)
    state = (lok, cntlo, hik, cnthi)

    c0 = mu + _Z_INIT * sig
    k0 = jnp.clip(_fold(c0), lok + 1, hik - 1)
    cnt0 = count(_unfold(k0))
    take0 = cnt0 >= TOPK
    state = (jnp.where(take0, k0, lok), jnp.where(take0, cnt0, cntlo),
             jnp.where(take0, hik, k0), jnp.where(take0, cnthi, cnt0))
    tgt = jnp.where(cnt0 < TOPK, jnp.int32(TOPK + 12), jnp.int32(TOPK - 8))
    c1 = _unfold(k0) + (cnt0 - tgt).astype(jnp.float32) * sig * _SLOPE
    done = state[1] == TOPK
    k1 = jnp.clip(_fold(c1), state[0] + 1, state[2] - 1)
    state = update(state, jnp.where(done, state[0], k1))

    def live_mask(st):
        lok, cntlo, hik, cnthi = st
        return ((cntlo != TOPK) & (cntlo - TOPK > _ASC) & ((lok + 1) < hik))

    def falsi_body(i, st):
        lok, cntlo, hik, cnthi = st
        lo_f, hi_f = _unfold(lok), _unfold(hik)
        frac = (cntlo.astype(jnp.float32) - _FRAC_TGT) / (
            (cntlo - cnthi).astype(jnp.float32))
        interpk = _fold(lo_f + (hi_f - lo_f) * jnp.clip(frac, 0.0, 1.0))
        midk = lok + ((hik >> 1) - (lok >> 1))
        candk = jnp.where((i % 3) == 2, midk, interpk)
        candk = jnp.clip(candk, lok + 1, hik - 1)
        candk = jnp.where(live_mask(st), candk, lok)
        return update(st, candk)

    state = jax.lax.fori_loop(0, 5, falsi_body, state)

    def safe_cond(carry):
        st, it = carry
        return jnp.logical_and(it < 64, jnp.any(live_mask(st)))

    def safe_body(carry):
        st, it = carry
        lok, cntlo, hik, cnthi = st
        midk = jnp.clip(lok + ((hik >> 1) - (lok >> 1)), lok + 1, hik - 1)
        candk = jnp.where(live_mask(st), midk, lok)
        return update(st, candk), it + 1

    state, _ = jax.lax.while_loop(safe_cond, safe_body, (state, jnp.int32(0)))

    lok, cntlo, hik, cnthi = state
    rem = jnp.where(cntlo == TOPK, jnp.int32(0), cntlo - TOPK)
    m = _unfold(lok - 1)

    def asc_cond(carry):
        _, rem, it = carry
        return jnp.logical_and(it < 256, jnp.any(rem > 0))

    def asc_body(carry):
        m, rem, it = carry
        act = rem > 0
        mnew = jnp.min(jnp.where(sim > m, sim, jnp.float32(3.4e38)),
                       axis=1, keepdims=True)
        m = jnp.where(act, mnew, m)
        rem = jnp.where(act, rem - 1, rem)
        return m, rem, it + 1

    m, _, _ = jax.lax.while_loop(
        asc_cond, asc_body, (m, rem, jnp.int32(0)))

    tf = jnp.where(cntlo == TOPK, _unfold(lok), _unfold(_fold(m) + 1))
    cnt_v = count(tf)
    dni = (cnt_v >= TOPK).astype(jnp.int32)
    hif = tf

    def rep_cond(carry):
        _, dni, _, it = carry
        return jnp.logical_and(it < 64, jnp.any(dni == 0))

    def rep_body(carry):
        tf, dni, hif, it = carry
        dn = dni > 0
        m2 = jnp.max(jnp.where(sim < hif, sim, jnp.float32(-3.4e38)),
                     axis=1, keepdims=True)
        cnt2 = count(m2)
        fin = (~dn) & (cnt2 >= TOPK)
        tf = jnp.where(fin, m2, tf)
        dn = dn | fin
        adv = (~dn) & (cnt2 < TOPK)
        hif = jnp.where(adv, m2, hif)
        return tf, dn.astype(jnp.int32), hif, it + 1

    tf, _, _, _ = jax.lax.while_loop(
        rep_cond, rep_body, (tf, dni, hif, jnp.int32(0)))

    o_ref[0] = jnp.where(sim >= tf, sim, 0.0)


@jax.jit
def kernel(embedding_matrix, Wq, bq, Wk, bk):
    x = embedding_matrix
    q, k = pl.pallas_call(
        _proj_body,
        grid=(N // _PB,),
        in_specs=[
            pl.BlockSpec((_PB, E), lambda i: (i, 0)),
            pl.BlockSpec((E, E), lambda i: (0, 0)),
            pl.BlockSpec((1, E), lambda i: (0, 0)),
            pl.BlockSpec((E, E), lambda i: (0, 0)),
            pl.BlockSpec((1, E), lambda i: (0, 0)),
        ],
        out_specs=[
            pl.BlockSpec((_PB, E), lambda i: (i, 0)),
            pl.BlockSpec((_PB, E), lambda i: (i, 0)),
        ],
        out_shape=[
            jax.ShapeDtypeStruct((N, E), jnp.float32),
            jax.ShapeDtypeStruct((N, E), jnp.float32),
        ],
    )(x, Wq, bq.reshape(1, E), Wk, bk.reshape(1, E))

    qh = q.reshape(N, H, DH).transpose(1, 0, 2)
    kh = k.reshape(N, H, DH).transpose(1, 0, 2)

    out = pl.pallas_call(
        _attn_body,
        grid=(H, N // _RB),
        in_specs=[
            pl.BlockSpec((1, _RB, DH), lambda h, i: (h, i, 0)),
            pl.BlockSpec((1, N, DH), lambda h, i: (h, 0, 0)),
        ],
        out_specs=pl.BlockSpec((1, _RB, N), lambda h, i: (h, i, 0)),
        out_shape=jax.ShapeDtypeStruct((H, N, N), jnp.float32),
    )(qh, kh)
    return out
